# Initial kernel scaffold; baseline (speedup 1.0000x reference)
import jax
import jax.numpy as jnp
from jax.experimental import pallas as pl


def kernel(x, action, W1, b1, Wg, bg, Wm, bm, Wp1, bp1, Wp2, bp2, Wc1, bc1, Wc2, bc2):
    raise NotImplementedError("write your pallas kernel here")


# speedup vs baseline: 1.2026x; 1.2026x over previous
import jax
import jax.numpy as jnp
from jax.experimental import pallas as pl

_J = 200
_M = 50
_B = 64
_D = 32
_N = 2 + 3 * _J + _M + 3 * _J * _M + _J * _J

_INTERPRET = False


def _body(x3_ref, rcol_ref, vrep_ref, ctp_ref, edges_ref, act_ref,
          W1_ref, b1_ref, Wg_ref, bg_ref, Wmt_ref, bmt_ref, G_ref, bBm_ref,
          EH_ref, K_ref, bp2_ref, Wc1h_ref, wc1v_ref, bc1_ref, Wc2_ref,
          bc2_ref, logp_ref, ent_ref, v_ref):
    J, M, D = _J, _M, _D
    f32 = jnp.float32
    hi = jax.lax.Precision.HIGHEST

    x3 = x3_ref[0]
    h0 = jnp.maximum(jnp.dot(x3, W1_ref[...], precision=hi,
                             preferred_element_type=f32) + b1_ref[...], 0.0)
    e = edges_ref[0]
    hg = h0 + jnp.dot(e, h0, precision=hi, preferred_element_type=f32)
    h = jnp.maximum(jnp.dot(hg, Wg_ref[...], precision=hi,
                            preferred_element_type=f32) + bg_ref[...], 0.0)

    vrep = vrep_ref[0]
    me = jnp.maximum(vrep * Wmt_ref[...] + bmt_ref[...], 0.0)
    Bm = jnp.dot(me, G_ref[...], precision=hi,
                 preferred_element_type=f32) + bBm_ref[...]

    ctp = ctp_ref[0]
    hctp = jnp.concatenate([h, ctp], axis=1)
    Zp = jnp.dot(hctp, EH_ref[...], preferred_element_type=f32)
    Z = jnp.maximum(Zp + Bm, 0.0)
    sc = jnp.dot(Z, K_ref[...], preferred_element_type=f32) + bp2_ref[0, 0]

    c = ctp[:, 0:M]
    rcol = rcol_ref[0]
    mask = jnp.logical_or(rcol == 0.0, c == 0.0)
    scm = jnp.where(mask, -100000000.0, sc)
    mx = jnp.max(scm, keepdims=True)
    ee = jnp.exp(scm - mx)
    se = jnp.sum(ee, keepdims=True)
    logz = mx + jnp.log(se)

    a = act_ref[0, 0, 0]
    idx = (jax.lax.broadcasted_iota(jnp.int32, (J, M), 0) * M
           + jax.lax.broadcasted_iota(jnp.int32, (J, M), 1))
    sc_a = jnp.sum(jnp.where(idx == a, scm, 0.0), keepdims=True)
    logp_ref[0] = sc_a - logz
    ent_ref[0] = logz - jnp.sum(ee * scm, keepdims=True) / se

    mh = jnp.mean(h, axis=0, keepdims=True)
    mv = jnp.sum(vrep, keepdims=True) / (M * D)
    gv = jnp.maximum(jnp.dot(mh, Wc1h_ref[...], precision=hi,
                             preferred_element_type=f32)
                     + mv * wc1v_ref[...] + bc1_ref[...], 0.0)
    v = jnp.dot(gv, Wc2_ref[...], precision=hi, preferred_element_type=f32)
    v_ref[0] = v + bc2_ref[...]


def kernel(x, action, W1, b1, Wg, bg, Wm, bm, Wp1, bp1, Wp2, bp2, Wc1, bc1,
           Wc2, bc2):
    J, M, B, D = _J, _M, _B, _D
    f32 = jnp.float32

    o = 2
    s = x[:, o:o + J]; o += J
    r = x[:, o:o + J]; o += J
    tct = x[:, o:o + J]; o += J
    vct = x[:, o:o + M]; o += M
    compat = x[:, o:o + J * M].reshape(B, J, M); o += J * M
    tc = x[:, o:o + J * M].reshape(B, J, M); o += J * M
    pc = x[:, o:o + J * M].reshape(B, J, M); o += J * M
    edges = x[:, o:o + J * J].reshape(B, J, J)

    x3p = jnp.concatenate(
        [jnp.stack([s, r, tct], axis=-1),
         jnp.zeros((B, J, 5), f32)], axis=-1)
    rcol = r[:, :, None]
    vrep = jnp.repeat(vct, D, axis=1)[:, None, :]
    ctp = jnp.concatenate(
        [compat, tc, pc, jnp.zeros((B, J, 2), f32)], axis=-1)
    act3 = action.astype(jnp.int32).reshape(B, 1, 1)

    W1p = jnp.concatenate([W1, jnp.zeros((5, D), f32)], axis=0)
    wc, wt, wp = Wp1[2 * D], Wp1[2 * D + 1], Wp1[2 * D + 2]
    eyeM = jnp.eye(M, dtype=f32)
    EH = jnp.concatenate([
        jnp.tile(Wp1[:D], (1, M)),
        jnp.kron(eyeM, wc[None, :]),
        jnp.kron(eyeM, wt[None, :]),
        jnp.kron(eyeM, wp[None, :]),
        jnp.zeros((2, M * D), f32),
    ], axis=0)
    G = jnp.kron(eyeM, Wp1[D:2 * D])
    K = jnp.kron(eyeM, Wp2)
    Wmt = jnp.tile(Wm[0], M)[None, :]
    bmt = jnp.tile(bm, M)[None, :]
    bBm = jnp.tile(bp1, M)[None, :]
    Wc1h = Wc1[:D]
    wc1v = Wc1[D][None, :]

    row2 = lambda v: v[None, :]
    grid = (B,)
    bs = pl.BlockSpec
    per_b = lambda shape: bs((1,) + shape, lambda i: (i, 0, 0))
    full2 = lambda a: bs(a.shape, lambda i: (0, 0))

    in_specs = [
        per_b((J, 8)), per_b((J, 1)), per_b((1, M * D)), per_b((J, 152)),
        per_b((J, J)), per_b((1, 1)),
    ]
    weights = [W1p, row2(b1), Wg, row2(bg), Wmt, bmt, G, bBm, EH, K,
               bp2.reshape(1, 1), Wc1h, wc1v, row2(bc1), Wc2,
               bc2.reshape(1, 1)]
    in_specs += [full2(w) for w in weights]

    out_shape = [jax.ShapeDtypeStruct((B, 1, 1), f32)] * 3
    out_specs = [per_b((1, 1))] * 3

    logp, ent, v = pl.pallas_call(
        _body,
        grid=grid,
        in_specs=in_specs,
        out_specs=out_specs,
        out_shape=out_shape,
        interpret=_INTERPRET,
    )(x3p, rcol, vrep, ctp, edges, act3, *weights)

    return (action, logp.reshape(B), ent.reshape(B), v.reshape(B))


# speedup vs baseline: 2.5734x; 2.1399x over previous
import jax
import jax.numpy as jnp
from jax.experimental import pallas as pl

_J = 200
_M = 50
_B = 64
_D = 32
_N = 2 + 3 * _J + _M + 3 * _J * _M + _J * _J

_INTERPRET = False


def _body(x3_ref, rcol_ref, vct_ref, ctp_ref, edges_ref, act_ref,
          W1_ref, b1_ref, Wg_ref, bg_ref, WmC_ref, bmC_ref, Wp1mT_ref,
          R_ref, bBm_ref, EH_ref, K_ref, bp2_ref, Wc1h_ref, wc1v_ref,
          bc1_ref, Wc2_ref, bc2_ref, logp_ref, ent_ref, v_ref):
    J, M, D = _J, _M, _D
    f32 = jnp.float32
    hi = jax.lax.Precision.HIGHEST

    x3 = x3_ref[0]
    h0 = jnp.maximum(jnp.dot(x3, W1_ref[...], precision=hi,
                             preferred_element_type=f32) + b1_ref[...], 0.0)
    e = edges_ref[0]
    hg = h0 + jnp.dot(e, h0, precision=hi, preferred_element_type=f32)
    h = jnp.maximum(jnp.dot(hg, Wg_ref[...], precision=hi,
                            preferred_element_type=f32) + bg_ref[...], 0.0)

    vctr = vct_ref[0]
    meT = jnp.maximum(WmC_ref[...] * vctr + bmC_ref[...], 0.0)
    Bm2T = jnp.dot(Wp1mT_ref[...], meT, precision=hi,
                   preferred_element_type=f32)
    E2 = jnp.dot(Bm2T, R_ref[...], precision=hi,
                 preferred_element_type=f32)
    i0 = jax.lax.broadcasted_iota(jnp.int32, (D, M * D), 0)
    i1 = jax.lax.broadcasted_iota(jnp.int32, (D, M * D), 1)
    Bm = jnp.sum(jnp.where((i1 & (D - 1)) == i0, E2, 0.0),
                 axis=0, keepdims=True) + bBm_ref[...]

    ctp = ctp_ref[0]
    hctp = jnp.concatenate([h, ctp], axis=1)
    Zp = jnp.dot(hctp, EH_ref[...], preferred_element_type=f32)
    Z = jnp.maximum(Zp + Bm, 0.0)
    sc = jnp.dot(Z, K_ref[...], preferred_element_type=f32) + bp2_ref[0, 0]

    c = ctp[:, 0:M]
    rcol = rcol_ref[0]
    mask = jnp.logical_or(rcol == 0.0, c == 0.0)
    scm = jnp.where(mask, -100000000.0, sc)
    mx = jnp.max(scm, keepdims=True)
    ee = jnp.exp(scm - mx)
    se = jnp.sum(ee, keepdims=True)
    logz = mx + jnp.log(se)

    a = act_ref[0, 0, 0]
    idx = (jax.lax.broadcasted_iota(jnp.int32, (J, M), 0) * M
           + jax.lax.broadcasted_iota(jnp.int32, (J, M), 1))
    sc_a = jnp.sum(jnp.where(idx == a, scm, 0.0), keepdims=True)
    logp_ref[0] = sc_a - logz
    ent_ref[0] = logz - jnp.sum(ee * scm, keepdims=True) / se

    mh = jnp.mean(h, axis=0, keepdims=True)
    mv = jnp.sum(vctr, keepdims=True) / M
    gv = jnp.maximum(jnp.dot(mh, Wc1h_ref[...], precision=hi,
                             preferred_element_type=f32)
                     + mv * wc1v_ref[...] + bc1_ref[...], 0.0)
    v = jnp.dot(gv, Wc2_ref[...], precision=hi, preferred_element_type=f32)
    v_ref[0] = v + bc2_ref[...]


def kernel(x, action, W1, b1, Wg, bg, Wm, bm, Wp1, bp1, Wp2, bp2, Wc1, bc1,
           Wc2, bc2):
    J, M, B, D = _J, _M, _B, _D
    f32 = jnp.float32

    o = 2
    s = x[:, o:o + J]; o += J
    r = x[:, o:o + J]; o += J
    tct = x[:, o:o + J]; o += J
    vct = x[:, o:o + M]; o += M
    compat = x[:, o:o + J * M].reshape(B, J, M); o += J * M
    tc = x[:, o:o + J * M].reshape(B, J, M); o += J * M
    pc = x[:, o:o + J * M].reshape(B, J, M); o += J * M
    edges = x[:, o:o + J * J].reshape(B, J, J)

    x3p = jnp.concatenate(
        [jnp.stack([s, r, tct], axis=-1),
         jnp.zeros((B, J, 5), f32)], axis=-1)
    rcol = r[:, :, None]
    vct3 = vct[:, None, :]
    ctp = jnp.concatenate(
        [compat, tc, pc, jnp.zeros((B, J, 2), f32)], axis=-1)
    act3 = action.astype(jnp.int32).reshape(B, 1, 1)

    W1p = jnp.concatenate([W1, jnp.zeros((5, D), f32)], axis=0)
    wc, wt, wp = Wp1[2 * D], Wp1[2 * D + 1], Wp1[2 * D + 2]
    eyeM = jnp.eye(M, dtype=f32)
    EH = jnp.concatenate([
        jnp.tile(Wp1[:D], (1, M)),
        jnp.kron(eyeM, wc[None, :]),
        jnp.kron(eyeM, wt[None, :]),
        jnp.kron(eyeM, wp[None, :]),
        jnp.zeros((2, M * D), f32),
    ], axis=0)
    K = jnp.kron(eyeM, Wp2)
    R = jnp.kron(eyeM, jnp.ones((1, D), f32))
    bBm = jnp.tile(bp1, M)[None, :]
    Wc1h = Wc1[:D]
    wc1v = Wc1[D][None, :]

    row2 = lambda v: v[None, :]
    grid = (B,)
    bs = pl.BlockSpec
    per_b = lambda shape: bs((1,) + shape, lambda i: (i, 0, 0))
    full2 = lambda a: bs(a.shape, lambda i: (0, 0))

    in_specs = [
        per_b((J, 8)), per_b((J, 1)), per_b((1, M)), per_b((J, 152)),
        per_b((J, J)), per_b((1, 1)),
    ]
    weights = [W1p, row2(b1), Wg, row2(bg), Wm[0][:, None], bm[:, None],
               Wp1[D:2 * D].T, R, bBm, EH, K,
               bp2.reshape(1, 1), Wc1h, wc1v, row2(bc1), Wc2,
               bc2.reshape(1, 1)]
    in_specs += [full2(w) for w in weights]

    out_shape = [jax.ShapeDtypeStruct((B, 1, 1), f32)] * 3
    out_specs = [per_b((1, 1))] * 3

    logp, ent, v = pl.pallas_call(
        _body,
        grid=grid,
        in_specs=in_specs,
        out_specs=out_specs,
        out_shape=out_shape,
        interpret=_INTERPRET,
    )(x3p, rcol, vct3, ctp, edges, act3, *weights)

    return (action, logp.reshape(B), ent.reshape(B), v.reshape(B))


# speedup vs baseline: 2.6232x; 1.0194x over previous
import jax
import jax.numpy as jnp
from jax.experimental import pallas as pl

_J = 200
_M = 50
_B = 64
_D = 32
_N = 2 + 3 * _J + _M + 3 * _J * _M + _J * _J

_INTERPRET = False
_SPB = 8


def _body(x3_ref, rcol_ref, vct_ref, ctp_ref, edges_ref, act_ref,
          W1_ref, b1_ref, Wg_ref, bg_ref, WmC_ref, bmC_ref, Wp1mT_ref,
          R_ref, bBm_ref, EH_ref, K_ref, bp2_ref, Wc1h_ref, wc1v_ref,
          bc1_ref, Wc2_ref, bc2_ref, logp_ref, ent_ref, v_ref):
    for k in range(_SPB):
        _one_sample(k, x3_ref, rcol_ref, vct_ref, ctp_ref, edges_ref,
                    act_ref, W1_ref, b1_ref, Wg_ref, bg_ref, WmC_ref,
                    bmC_ref, Wp1mT_ref, R_ref, bBm_ref, EH_ref, K_ref,
                    bp2_ref, Wc1h_ref, wc1v_ref, bc1_ref, Wc2_ref, bc2_ref,
                    logp_ref, ent_ref, v_ref)


def _one_sample(k, x3_ref, rcol_ref, vct_ref, ctp_ref, edges_ref, act_ref,
                W1_ref, b1_ref, Wg_ref, bg_ref, WmC_ref, bmC_ref, Wp1mT_ref,
                R_ref, bBm_ref, EH_ref, K_ref, bp2_ref, Wc1h_ref, wc1v_ref,
                bc1_ref, Wc2_ref, bc2_ref, logp_ref, ent_ref, v_ref):
    J, M, D = _J, _M, _D
    f32 = jnp.float32
    hi = jax.lax.Precision.HIGHEST

    x3 = x3_ref[k]
    h0 = jnp.maximum(jnp.dot(x3, W1_ref[...], precision=hi,
                             preferred_element_type=f32) + b1_ref[...], 0.0)
    e = edges_ref[k]
    hg = h0 + jnp.dot(e, h0, precision=hi, preferred_element_type=f32)
    h = jnp.maximum(jnp.dot(hg, Wg_ref[...], precision=hi,
                            preferred_element_type=f32) + bg_ref[...], 0.0)

    vctr = vct_ref[k]
    meT = jnp.maximum(WmC_ref[...] * vctr + bmC_ref[...], 0.0)
    Bm2T = jnp.dot(Wp1mT_ref[...], meT, precision=hi,
                   preferred_element_type=f32)
    E2 = jnp.dot(Bm2T, R_ref[...], precision=hi,
                 preferred_element_type=f32)
    i0 = jax.lax.broadcasted_iota(jnp.int32, (D, M * D), 0)
    i1 = jax.lax.broadcasted_iota(jnp.int32, (D, M * D), 1)
    Bm = jnp.sum(jnp.where((i1 & (D - 1)) == i0, E2, 0.0),
                 axis=0, keepdims=True) + bBm_ref[...]

    ctp = ctp_ref[k]
    hctp = jnp.concatenate([h, ctp], axis=1)
    Zp = jnp.dot(hctp, EH_ref[...], preferred_element_type=f32)
    Z = jnp.maximum(Zp + Bm, 0.0)
    sc = jnp.dot(Z, K_ref[...], preferred_element_type=f32) + bp2_ref[0, 0]

    c = ctp[:, 0:M]
    rcol = rcol_ref[k]
    mask = jnp.logical_or(rcol == 0.0, c == 0.0)
    scm = jnp.where(mask, -100000000.0, sc)
    mx = jnp.max(scm, keepdims=True)
    ee = jnp.exp(scm - mx)
    se = jnp.sum(ee, keepdims=True)
    logz = mx + jnp.log(se)

    a = act_ref[k, 0, 0]
    idx = (jax.lax.broadcasted_iota(jnp.int32, (J, M), 0) * M
           + jax.lax.broadcasted_iota(jnp.int32, (J, M), 1))
    sc_a = jnp.sum(jnp.where(idx == a, scm, 0.0), keepdims=True)
    logp_ref[k] = sc_a - logz
    ent_ref[k] = logz - jnp.sum(ee * scm, keepdims=True) / se

    mh = jnp.mean(h, axis=0, keepdims=True)
    mv = jnp.sum(vctr, keepdims=True) / M
    gv = jnp.maximum(jnp.dot(mh, Wc1h_ref[...], precision=hi,
                             preferred_element_type=f32)
                     + mv * wc1v_ref[...] + bc1_ref[...], 0.0)
    v = jnp.dot(gv, Wc2_ref[...], precision=hi, preferred_element_type=f32)
    v_ref[k] = v + bc2_ref[...]


def kernel(x, action, W1, b1, Wg, bg, Wm, bm, Wp1, bp1, Wp2, bp2, Wc1, bc1,
           Wc2, bc2):
    J, M, B, D = _J, _M, _B, _D
    f32 = jnp.float32

    o = 2
    s = x[:, o:o + J]; o += J
    r = x[:, o:o + J]; o += J
    tct = x[:, o:o + J]; o += J
    vct = x[:, o:o + M]; o += M
    compat = x[:, o:o + J * M].reshape(B, J, M); o += J * M
    tc = x[:, o:o + J * M].reshape(B, J, M); o += J * M
    pc = x[:, o:o + J * M].reshape(B, J, M); o += J * M
    edges = x[:, o:o + J * J].reshape(B, J, J)

    x3p = jnp.concatenate(
        [jnp.stack([s, r, tct], axis=-1),
         jnp.zeros((B, J, 5), f32)], axis=-1)
    rcol = r[:, :, None]
    vct3 = vct[:, None, :]
    ctp = jnp.concatenate(
        [compat, tc, pc, jnp.zeros((B, J, 2), f32)], axis=-1)
    act3 = action.astype(jnp.int32).reshape(B, 1, 1)

    W1p = jnp.concatenate([W1, jnp.zeros((5, D), f32)], axis=0)
    wc, wt, wp = Wp1[2 * D], Wp1[2 * D + 1], Wp1[2 * D + 2]
    eyeM = jnp.eye(M, dtype=f32)
    EH = jnp.concatenate([
        jnp.tile(Wp1[:D], (1, M)),
        jnp.kron(eyeM, wc[None, :]),
        jnp.kron(eyeM, wt[None, :]),
        jnp.kron(eyeM, wp[None, :]),
        jnp.zeros((2, M * D), f32),
    ], axis=0)
    K = jnp.kron(eyeM, Wp2)
    R = jnp.kron(eyeM, jnp.ones((1, D), f32))
    bBm = jnp.tile(bp1, M)[None, :]
    Wc1h = Wc1[:D]
    wc1v = Wc1[D][None, :]

    row2 = lambda v: v[None, :]
    grid = (B // _SPB,)
    bs = pl.BlockSpec
    per_b = lambda shape: bs((_SPB,) + shape, lambda i: (i, 0, 0))
    full2 = lambda a: bs(a.shape, lambda i: (0, 0))

    in_specs = [
        per_b((J, 8)), per_b((J, 1)), per_b((1, M)), per_b((J, 152)),
        per_b((J, J)), per_b((1, 1)),
    ]
    weights = [W1p, row2(b1), Wg, row2(bg), Wm[0][:, None], bm[:, None],
               Wp1[D:2 * D].T, R, bBm, EH, K,
               bp2.reshape(1, 1), Wc1h, wc1v, row2(bc1), Wc2,
               bc2.reshape(1, 1)]
    in_specs += [full2(w) for w in weights]

    out_shape = [jax.ShapeDtypeStruct((B, 1, 1), f32)] * 3
    out_specs = [per_b((1, 1))] * 3

    logp, ent, v = pl.pallas_call(
        _body,
        grid=grid,
        in_specs=in_specs,
        out_specs=out_specs,
        out_shape=out_shape,
        interpret=_INTERPRET,
    )(x3p, rcol, vct3, ctp, edges, act3, *weights)

    return (action, logp.reshape(B), ent.reshape(B), v.reshape(B))


# speedup vs baseline: 4.0143x; 1.5303x over previous
import jax
import jax.numpy as jnp
from jax.experimental import pallas as pl

_J = 200
_M = 50
_B = 64
_D = 32

_INTERPRET = False
_SPB = 8


def _body(str_ref, vct_ref, ctp_ref, edges_ref, act_ref,
          W1R_ref, b1_ref, Wg_ref, bg_ref, WmC_ref, bmC_ref, Wp1mT_ref,
          R_ref, bBm_ref, EH_ref, K_ref, bp2_ref, Wc1h_ref, wc1v_ref,
          bc1_ref, Wc2_ref, bc2_ref, logp_ref, ent_ref, v_ref):
    for k in range(_SPB):
        _one_sample(k, str_ref, vct_ref, ctp_ref, edges_ref, act_ref,
                    W1R_ref, b1_ref, Wg_ref, bg_ref, WmC_ref, bmC_ref,
                    Wp1mT_ref, R_ref, bBm_ref, EH_ref, K_ref, bp2_ref,
                    Wc1h_ref, wc1v_ref, bc1_ref, Wc2_ref, bc2_ref,
                    logp_ref, ent_ref, v_ref)


def _one_sample(k, str_ref, vct_ref, ctp_ref, edges_ref, act_ref,
                W1R_ref, b1_ref, Wg_ref, bg_ref, WmC_ref, bmC_ref,
                Wp1mT_ref, R_ref, bBm_ref, EH_ref, K_ref, bp2_ref,
                Wc1h_ref, wc1v_ref, bc1_ref, Wc2_ref, bc2_ref,
                logp_ref, ent_ref, v_ref):
    J, M, D = _J, _M, _D
    f32 = jnp.float32
    bf16 = jnp.bfloat16
    hi = jax.lax.Precision.HIGHEST

    s = str_ref[k, 0]
    r = str_ref[k, 1]
    tct = str_ref[k, 2]
    h0 = jnp.maximum(s * W1R_ref[0:1, :] + r * W1R_ref[1:2, :]
                     + tct * W1R_ref[2:3, :] + b1_ref[...], 0.0)
    e = edges_ref[k]
    hg = h0 + jnp.dot(e, h0.astype(bf16), preferred_element_type=f32)
    h = jnp.maximum(jnp.dot(hg, Wg_ref[...], precision=hi,
                            preferred_element_type=f32) + bg_ref[...], 0.0)

    vctr = vct_ref[k]
    meT = jnp.maximum(WmC_ref[...] * vctr + bmC_ref[...], 0.0)
    Bm2T = jnp.dot(Wp1mT_ref[...], meT, precision=hi,
                   preferred_element_type=f32)
    E2 = jnp.dot(Bm2T.astype(bf16), R_ref[...],
                 preferred_element_type=f32)
    i0 = jax.lax.broadcasted_iota(jnp.int32, (D, M * D), 0)
    i1 = jax.lax.broadcasted_iota(jnp.int32, (D, M * D), 1)
    Bm = jnp.sum(jnp.where((i1 & (D - 1)) == i0, E2, 0.0),
                 axis=0, keepdims=True) + bBm_ref[...]

    c = ctp_ref[k, 0]
    t = ctp_ref[k, 1]
    p = ctp_ref[k, 2]
    hctp = jnp.concatenate([h.astype(bf16), c, t, p], axis=1)
    Zp = jnp.dot(hctp, EH_ref[...], preferred_element_type=f32)
    Z = jnp.maximum(Zp + Bm, 0.0)
    sc = jnp.dot(Z.astype(bf16), K_ref[...],
                 preferred_element_type=f32) + bp2_ref[0, 0]

    valid = r * c.astype(f32)
    scm = jnp.where(valid == 0.0, -100000000.0, sc)
    mx = jnp.max(scm, keepdims=True)
    ee = jnp.exp(scm - mx)
    se = jnp.sum(ee, keepdims=True)
    logz = mx + jnp.log(se)

    a = act_ref[k, 0, 0]
    idx = (jax.lax.broadcasted_iota(jnp.int32, (J, M), 0) * M
           + jax.lax.broadcasted_iota(jnp.int32, (J, M), 1))
    sc_a = jnp.sum(jnp.where(idx == a, scm, 0.0), keepdims=True)
    logp_ref[k] = sc_a - logz
    ent_ref[k] = logz - jnp.sum(ee * scm, keepdims=True) / se

    mh = jnp.mean(h, axis=0, keepdims=True)
    mv = jnp.sum(vctr, keepdims=True) / M
    gv = jnp.maximum(jnp.dot(mh, Wc1h_ref[...], precision=hi,
                             preferred_element_type=f32)
                     + mv * wc1v_ref[...] + bc1_ref[...], 0.0)
    v = jnp.dot(gv, Wc2_ref[...], precision=hi, preferred_element_type=f32)
    v_ref[k] = v + bc2_ref[...]


def kernel(x, action, W1, b1, Wg, bg, Wm, bm, Wp1, bp1, Wp2, bp2, Wc1, bc1,
           Wc2, bc2):
    J, M, B, D = _J, _M, _B, _D
    f32 = jnp.float32
    bf16 = jnp.bfloat16

    str3 = x[:, 2:2 + 3 * J].reshape(B, 3, J, 1)
    vct3 = x[:, 2 + 3 * J:2 + 3 * J + M][:, None, :]
    o = 2 + 3 * J + M
    ctp3 = x[:, o:o + 3 * J * M].reshape(B, 3, J, M).astype(bf16)
    edges = x[:, o + 3 * J * M:].reshape(B, J, J).astype(bf16)
    act3 = action.astype(jnp.int32).reshape(B, 1, 1)

    wc, wt, wp = Wp1[2 * D], Wp1[2 * D + 1], Wp1[2 * D + 2]
    eyeM = jnp.eye(M, dtype=f32)
    EH = jnp.concatenate([
        jnp.tile(Wp1[:D], (1, M)),
        jnp.kron(eyeM, wc[None, :]),
        jnp.kron(eyeM, wt[None, :]),
        jnp.kron(eyeM, wp[None, :]),
    ], axis=0).astype(bf16)
    K = jnp.kron(eyeM, Wp2).astype(bf16)
    R = jnp.kron(eyeM, jnp.ones((1, D), f32)).astype(bf16)
    bBm = jnp.tile(bp1, M)[None, :]
    Wc1h = Wc1[:D]
    wc1v = Wc1[D][None, :]

    row2 = lambda v: v[None, :]
    grid = (B // _SPB,)
    bs = pl.BlockSpec
    per_b = lambda shape: bs((_SPB,) + shape,
                             lambda i: (i,) + (0,) * len(shape))
    full2 = lambda a: bs(a.shape, lambda i: (0, 0))

    in_specs = [
        per_b((3, J, 1)), per_b((1, M)), per_b((3, J, M)),
        per_b((J, J)), per_b((1, 1)),
    ]
    weights = [W1[:3], row2(b1), Wg, row2(bg), Wm[0][:, None], bm[:, None],
               Wp1[D:2 * D].T, R, bBm, EH, K,
               bp2.reshape(1, 1), Wc1h, wc1v, row2(bc1), Wc2,
               bc2.reshape(1, 1)]
    in_specs += [full2(w) for w in weights]

    out_shape = [jax.ShapeDtypeStruct((B, 1, 1), f32)] * 3
    out_specs = [per_b((1, 1))] * 3

    logp, ent, v = pl.pallas_call(
        _body,
        grid=grid,
        in_specs=in_specs,
        out_specs=out_specs,
        out_shape=out_shape,
        interpret=_INTERPRET,
    )(str3, vct3, ctp3, edges, act3, *weights)

    return (action, logp.reshape(B), ent.reshape(B), v.reshape(B))
